# Initial kernel scaffold; baseline (speedup 1.0000x reference)
import jax
import jax.numpy as jnp
from jax.experimental import pallas as pl


def kernel(inputs, kernel, u):
    raise NotImplementedError("write your pallas kernel here")


# speedup vs baseline: 11.7986x; 11.7986x over previous
import functools

import jax
import jax.numpy as jnp
from jax import lax
from jax.experimental import pallas as pl
from jax.experimental.pallas import tpu as pltpu
from jax.experimental.pallas import tpu_sc as plsc

VOCAB = 100000
IN_DIM = 26
FILTERS = 32
BATCH = 1024
D = IN_DIM * FILTERS
N_IDX = BATCH * IN_DIM
NW = 32
ROWS_PER_W = N_IDX // NW
CHUNK = 64
N_CHUNKS = ROWS_PER_W // CHUNK
SIG_BLK = 2000
SIG_GRID = VOCAB // SIG_BLK
_EPS = 1e-12


def _sigma_body(x_ref, u_ref, out_ref, wuf_ref, a2_ref):
    i = pl.program_id(0)

    @pl.when(i == 0)
    def _init():
        a2_ref[0] = 0.0
        wuf_ref[...] = jnp.zeros_like(wuf_ref)

    x = x_ref[...]
    a = lax.dot_general(x, u_ref[...], (((1,), (0,)), ((), ())),
                        preferred_element_type=jnp.float32)
    a2_ref[0] += jnp.sum(a * a)
    wuf_ref[...] += lax.dot_general(x, a, (((0,), (0,)), ((), ())),
                                    preferred_element_type=jnp.float32)

    @pl.when(i == SIG_GRID - 1)
    def _finalize():
        k_i = lax.broadcasted_iota(jnp.int32, (D, IN_DIM), 0)
        c_i = lax.broadcasted_iota(jnp.int32, (D, IN_DIM), 1)
        e = jnp.sum(jnp.where(k_i // FILTERS == c_i, wuf_ref[...], 0.0),
                    axis=1, keepdims=True)
        kj = lax.broadcasted_iota(jnp.int32, (D, FILTERS), 0)
        jj = lax.broadcasted_iota(jnp.int32, (D, FILTERS), 1)
        r = (kj % FILTERS == jj).astype(jnp.float32)
        wu = lax.dot_general(e, r, (((0,), (0,)), ((), ())),
                             preferred_element_type=jnp.float32)
        n1 = jnp.sqrt(a2_ref[0])
        n2 = jnp.sqrt(jnp.sum(wu * wu)) / (n1 + _EPS)
        out_ref[0, 0] = (n2 + _EPS) / (n2 * n2)


def _sigma_inv(tbl, u_mat):
    return pl.pallas_call(
        _sigma_body,
        grid=(SIG_GRID,),
        in_specs=[
            pl.BlockSpec((SIG_BLK, D), lambda i: (i, 0)),
            pl.BlockSpec((D, IN_DIM), lambda i: (0, 0)),
        ],
        out_specs=pl.BlockSpec(memory_space=pltpu.SMEM),
        out_shape=jax.ShapeDtypeStruct((1, 1), jnp.float32),
        scratch_shapes=[
            pltpu.VMEM((D, IN_DIM), jnp.float32),
            pltpu.SMEM((1,), jnp.float32),
        ],
    )(tbl, u_mat)


def _gather_body(tbl_hbm, idx_hbm, scale_hbm, out_hbm,
                 idx_v, scale_v, rows0, rows1,
                 gsem0, gsem1, ssem0, ssem1):
    wid = lax.axis_index("s") * 2 + lax.axis_index("c")
    pltpu.sync_copy(idx_hbm.at[wid], idx_v)
    pltpu.sync_copy(scale_hbm, scale_v)
    sv = scale_v[...]
    bufs = (rows0, rows1)
    gsems = (gsem0, gsem1)
    ssems = (ssem0, ssem1)

    def _scale(buf):
        def _row(rr, carry):
            for s in range(D // 16):
                sl = (rr, pl.ds(s * 16, 16))
                buf[sl] = buf[sl] * sv
            return carry
        lax.fori_loop(0, CHUNK, _row, 0)

    gathers = [None] * N_CHUNKS
    scatters = [None] * N_CHUNKS
    gathers[0] = pltpu.async_copy(tbl_hbm.at[idx_v.at[0]], bufs[0], gsems[0])
    for k in range(N_CHUNKS):
        b = k % 2
        if k + 1 < N_CHUNKS:
            nb = (k + 1) % 2
            if k >= 1:
                scatters[k - 1].wait()
            gathers[k + 1] = pltpu.async_copy(
                tbl_hbm.at[idx_v.at[k + 1]], bufs[nb], gsems[nb])
        gathers[k].wait()
        _scale(bufs[b])
        base = wid * ROWS_PER_W + k * CHUNK
        scatters[k] = pltpu.async_copy(
            bufs[b], out_hbm.at[pl.ds(base, CHUNK)], ssems[b])
    scatters[N_CHUNKS - 2].wait()
    scatters[N_CHUNKS - 1].wait()


def _gather_scale(tbl, idx2d, scale16):
    mesh = plsc.VectorSubcoreMesh(core_axis_name="c", subcore_axis_name="s")
    fn = functools.partial(
        pl.kernel,
        mesh=mesh,
        compiler_params=pltpu.CompilerParams(use_tc_tiling_on_sc=False),
        out_type=jax.ShapeDtypeStruct((N_IDX, D), jnp.float32),
        scratch_types=[
            pltpu.VMEM((N_CHUNKS, CHUNK), jnp.int32),
            pltpu.VMEM((16,), jnp.float32),
            pltpu.VMEM((CHUNK, D), jnp.float32),
            pltpu.VMEM((CHUNK, D), jnp.float32),
            pltpu.SemaphoreType.DMA,
            pltpu.SemaphoreType.DMA,
            pltpu.SemaphoreType.DMA,
            pltpu.SemaphoreType.DMA,
        ],
    )(_gather_body)
    return fn(tbl, idx2d, scale16)


def kernel(inputs, kernel, u):
    tbl = kernel.reshape(VOCAB, D)
    idx2d = inputs.reshape(NW, N_CHUNKS, CHUNK)
    ku = jnp.tile(u.reshape(FILTERS), IN_DIM)
    grp = (jnp.arange(D)[:, None] // FILTERS) == jnp.arange(IN_DIM)[None, :]
    u_mat = (ku[:, None] * grp).astype(jnp.float32)
    sig_inv = _sigma_inv(tbl, u_mat)
    scale16 = jnp.full((16,), sig_inv[0, 0], dtype=jnp.float32)
    out = _gather_scale(tbl, idx2d, scale16)
    return out.reshape(BATCH, 1, 1, 1, IN_DIM, 1, 1, IN_DIM, FILTERS)
